# baseline (speedup 1.0000x reference)
import functools

import jax
import jax.numpy as jnp
from jax import lax
from jax.experimental import pallas as pl
from jax.experimental.pallas import tpu as pltpu
from jax.experimental.pallas import tpu_sc as plsc

N = 10000
D = 128
E = 320000
G = 128
NC, NS = 2, 16
NW = NC * NS
CH = 128
EPT = E // NW
NCHUNK = 81
EPT_PAD = NCHUNK * CH
PAD_E = EPT_PAD - EPT
AGG_ROWS = 10040
WRT = 632
WRT_LAST = AGG_ROWS - 15 * WRT

RB = 2000
NBLK = N // RB


NBUF = 3


@functools.lru_cache(maxsize=None)
def _make_sc_agg():
    mesh = plsc.VectorSubcoreMesh(core_axis_name="c", subcore_axis_name="s")

    NG = NCHUNK // NBUF

    @functools.partial(
        pl.kernel,
        mesh=mesh,
        out_type=jax.ShapeDtypeStruct((NC, AGG_ROWS, D), jnp.float32),
        scratch_types=[
            pltpu.VMEM((2, NBUF, 2, CH), jnp.int32),
            pltpu.VMEM((CH, D), jnp.float32),
            pltpu.VMEM((CH, D), jnp.float32),
            pltpu.VMEM((CH, D), jnp.float32),
            pltpu.VMEM_SHARED((AGG_ROWS, D), jnp.float32),
            pltpu.SemaphoreType.DMA,
            pltpu.SemaphoreType.DMA,
            pltpu.SemaphoreType.DMA,
            pltpu.SemaphoreType.DMA,
            pltpu.SemaphoreType.DMA,
            pltpu.SemaphoreType.DMA,
            pltpu.SemaphoreType.DMA,
            pltpu.SemaphoreType.DMA,
        ],
    )
    def _sc_agg(h_hbm, epk_hbm, zinit_hbm, out_hbm,
                ebuf, r0, r1, r2, agg_sh, g0, g1, g2, s0, s1, s2, i0, i1):
        rbufs = [r0, r1, r2]
        gsems = [g0, g1, g2]
        ssems = [s0, s1, s2]
        isems = [i0, i1]
        c = lax.axis_index("c")
        s = lax.axis_index("s")
        wid = s * NC + c
        my_epk = epk_hbm.at[wid]
        icp = pltpu.async_copy(my_epk.at[pl.ds(0, NBUF)], ebuf.at[0],
                               isems[0])

        @pl.when(s < NS - 1)
        def _():
            pltpu.sync_copy(zinit_hbm, agg_sh.at[pl.ds(s * WRT, WRT)])

        @pl.when(s == NS - 1)
        def _():
            pltpu.sync_copy(zinit_hbm.at[pl.ds(0, WRT_LAST)],
                            agg_sh.at[pl.ds((NS - 1) * WRT, WRT_LAST)])

        icp.wait()
        plsc.subcore_barrier()

        def body(g, carry):
            slot = lax.rem(g, 2)
            for ss in range(2):
                @pl.when(slot == ss)
                def _():
                    @pl.when(g > 0)
                    def _():
                        for b in range(NBUF):
                            pltpu.make_async_copy(
                                rbufs[b], agg_sh.at[ebuf.at[1 - ss, b, 1]],
                                ssems[b]).wait()

                    @pl.when(g + 1 < NG)
                    def _():
                        pltpu.async_copy(
                            my_epk.at[pl.ds((g + 1) * NBUF, NBUF)],
                            ebuf.at[1 - ss], isems[1 - ss])

                    for b in range(NBUF):
                        pltpu.async_copy(h_hbm.at[ebuf.at[ss, b, 0]],
                                         rbufs[b], gsems[b])

                    for b in range(NBUF):
                        pltpu.make_async_copy(h_hbm.at[ebuf.at[ss, b, 0]],
                                              rbufs[b], gsems[b]).wait()
                        pltpu.async_copy(rbufs[b],
                                         agg_sh.at[ebuf.at[ss, b, 1]],
                                         ssems[b], add=True)

                    @pl.when(g + 1 < NG)
                    def _():
                        pltpu.make_async_copy(
                            my_epk.at[pl.ds((g + 1) * NBUF, NBUF)],
                            ebuf.at[1 - ss], isems[1 - ss]).wait()
            return carry

        lax.fori_loop(0, NG, body, 0)
        for b in range(NBUF):
            pltpu.make_async_copy(rbufs[b], agg_sh.at[ebuf.at[0, b, 1]],
                                  ssems[b]).wait()
        plsc.subcore_barrier()

        @pl.when(s < NS - 1)
        def _():
            pltpu.sync_copy(agg_sh.at[pl.ds(s * WRT, WRT)],
                            out_hbm.at[c].at[pl.ds(s * WRT, WRT)])

        @pl.when(s == NS - 1)
        def _():
            pltpu.sync_copy(agg_sh.at[pl.ds((NS - 1) * WRT, WRT_LAST)],
                            out_hbm.at[c].at[pl.ds((NS - 1) * WRT, WRT_LAST)])

    return _sc_agg


def _tc_layer1_body(part, h, w1, b1, w2, b2, bat, hout, pool):
    i = pl.program_id(0)
    agg = part[0] + part[1] + h[...]
    h1 = jnp.maximum(jnp.dot(agg, w1[...], preferred_element_type=jnp.float32)
                     + b1[...], 0.0)
    h2 = jnp.maximum(jnp.dot(h1, w2[...], preferred_element_type=jnp.float32)
                     + b2[...], 0.0)
    hout[...] = h2
    oh = (bat[0] == lax.broadcasted_iota(jnp.int32, (G, RB), 0)
          ).astype(jnp.float32)
    contrib = jnp.dot(oh, h2, preferred_element_type=jnp.float32)

    @pl.when(i == 0)
    def _init():
        pool[...] = jnp.zeros((G, G), jnp.float32)

    pool[...] += contrib


def _tc_layer2_body(part, h, w1, b1, w2, b2, bat, pool1, wp1, bp1, wp2, bp2,
                    ph_out, out2, pacc):
    i = pl.program_id(0)
    agg = part[0] + part[1] + h[...]
    h1 = jnp.maximum(jnp.dot(agg, w1[...], preferred_element_type=jnp.float32)
                     + b1[...], 0.0)
    h2 = jnp.maximum(jnp.dot(h1, w2[...], preferred_element_type=jnp.float32)
                     + b2[...], 0.0)
    oh = (bat[0] == lax.broadcasted_iota(jnp.int32, (G, RB), 0)
          ).astype(jnp.float32)
    contrib = jnp.dot(oh, h2, preferred_element_type=jnp.float32)

    @pl.when(i == 0)
    def _init():
        pacc[...] = jnp.zeros((G, G), jnp.float32)

    pacc[...] += contrib

    @pl.when(i == NBLK - 1)
    def _finish():
        ph = jnp.concatenate([pool1[...], pacc[...]], axis=-1)
        p = jnp.maximum(jnp.dot(ph, wp1[...],
                                preferred_element_type=jnp.float32)
                        + bp1[...], 0.0)
        out2[...] = jnp.dot(p, wp2[...],
                            preferred_element_type=jnp.float32) + bp2[...]
        ph_out[...] = ph


def _tc_layer1(part, h, w1, b1, w2, b2, bat3):
    return pl.pallas_call(
        _tc_layer1_body,
        grid=(NBLK,),
        in_specs=[
            pl.BlockSpec((2, RB, D), lambda i: (0, i, 0)),
            pl.BlockSpec((RB, D), lambda i: (i, 0)),
            pl.BlockSpec((D, D), lambda i: (0, 0)),
            pl.BlockSpec((1, D), lambda i: (0, 0)),
            pl.BlockSpec((D, D), lambda i: (0, 0)),
            pl.BlockSpec((1, D), lambda i: (0, 0)),
            pl.BlockSpec((1, 1, RB), lambda i: (i, 0, 0)),
        ],
        out_specs=[
            pl.BlockSpec((RB, D), lambda i: (i, 0)),
            pl.BlockSpec((G, G), lambda i: (0, 0)),
        ],
        out_shape=[
            jax.ShapeDtypeStruct((N, D), jnp.float32),
            jax.ShapeDtypeStruct((G, G), jnp.float32),
        ],
    )(part, h, w1, b1, w2, b2, bat3)


def _tc_layer2(part, h, w1, b1, w2, b2, bat3, pool1, wp1, bp1, wp2, bp2):
    return pl.pallas_call(
        _tc_layer2_body,
        grid=(NBLK,),
        in_specs=[
            pl.BlockSpec((2, RB, D), lambda i: (0, i, 0)),
            pl.BlockSpec((RB, D), lambda i: (i, 0)),
            pl.BlockSpec((D, D), lambda i: (0, 0)),
            pl.BlockSpec((1, D), lambda i: (0, 0)),
            pl.BlockSpec((D, D), lambda i: (0, 0)),
            pl.BlockSpec((1, D), lambda i: (0, 0)),
            pl.BlockSpec((1, 1, RB), lambda i: (i, 0, 0)),
            pl.BlockSpec((G, G), lambda i: (0, 0)),
            pl.BlockSpec((2 * D, D), lambda i: (0, 0)),
            pl.BlockSpec((1, D), lambda i: (0, 0)),
            pl.BlockSpec((D, D), lambda i: (0, 0)),
            pl.BlockSpec((1, D), lambda i: (0, 0)),
        ],
        out_specs=[
            pl.BlockSpec((G, 2 * D), lambda i: (0, 0)),
            pl.BlockSpec((G, D), lambda i: (0, 0)),
        ],
        out_shape=[
            jax.ShapeDtypeStruct((G, 2 * D), jnp.float32),
            jax.ShapeDtypeStruct((G, D), jnp.float32),
        ],
        scratch_shapes=[pltpu.VMEM((G, G), jnp.float32)],
    )(part, h, w1, b1, w2, b2, bat3, pool1, wp1, bp1, wp2, bp2)


def kernel(x, edge_index, batch, W1_0, b1_0, W2_0, b2_0, W1_1, b1_1,
           W2_1, b2_1, Wp1, bp1, Wp2, bp2):
    src = edge_index[0].astype(jnp.int32)
    dst = edge_index[1].astype(jnp.int32)

    ar = jnp.arange(PAD_E, dtype=jnp.int32)[None, :]
    w = jnp.arange(NW, dtype=jnp.int32)[:, None]
    pad_src = (w * 313 + ar) % N
    pad_dst = N + (w * 8 + ar) % (AGG_ROWS - N)
    srcp = jnp.concatenate([src.reshape(NW, EPT), pad_src],
                           axis=1).reshape(NW, NCHUNK, CH)
    dstp = jnp.concatenate([dst.reshape(NW, EPT), pad_dst],
                           axis=1).reshape(NW, NCHUNK, CH)
    epk = jnp.stack([srcp, dstp], axis=2)
    zinit = jnp.zeros((WRT, D), jnp.float32)
    bat3 = batch.astype(jnp.int32).reshape(NBLK, 1, RB)

    b1_0r, b2_0r = b1_0.reshape(1, D), b2_0.reshape(1, D)
    b1_1r, b2_1r = b1_1.reshape(1, D), b2_1.reshape(1, D)
    bp1r, bp2r = bp1.reshape(1, D), bp2.reshape(1, D)

    sc_agg = _make_sc_agg()
    part1 = sc_agg(x, epk, zinit)
    h1, pool1 = _tc_layer1(part1, x, W1_0, b1_0r, W2_0, b2_0r, bat3)
    part2 = sc_agg(h1, epk, zinit)
    pooled_h, pooled_h_p = _tc_layer2(part2, h1, W1_1, b1_1r, W2_1, b2_1r,
                                      bat3, pool1, Wp1, bp1r, Wp2, bp2r)
    return (pooled_h, pooled_h_p, x)


# speedup vs baseline: 1.0036x; 1.0036x over previous
import functools

import jax
import jax.numpy as jnp
from jax import lax
from jax.experimental import pallas as pl
from jax.experimental.pallas import tpu as pltpu
from jax.experimental.pallas import tpu_sc as plsc

N = 10000
D = 128
E = 320000
G = 128
NC, NS = 2, 16
NW = NC * NS
CH = 128
EPT = E // NW
NCHUNK = 81
EPT_PAD = NCHUNK * CH
PAD_E = EPT_PAD - EPT
AGG_ROWS = 10040
WRT = 632
WRT_LAST = AGG_ROWS - 15 * WRT

RB = 2000
NBLK = N // RB


NBUF = 3


@functools.lru_cache(maxsize=None)
def _make_sc_agg():
    mesh = plsc.VectorSubcoreMesh(core_axis_name="c", subcore_axis_name="s")

    NG = NCHUNK // NBUF

    @functools.partial(
        pl.kernel,
        mesh=mesh,
        out_type=jax.ShapeDtypeStruct((NC, AGG_ROWS, D), jnp.float32),
        scratch_types=[
            pltpu.VMEM((2, NBUF, 2, CH), jnp.int32),
            pltpu.VMEM((CH, D), jnp.float32),
            pltpu.VMEM((CH, D), jnp.float32),
            pltpu.VMEM((CH, D), jnp.float32),
            pltpu.VMEM_SHARED((AGG_ROWS, D), jnp.float32),
            pltpu.SemaphoreType.DMA,
            pltpu.SemaphoreType.DMA,
            pltpu.SemaphoreType.DMA,
            pltpu.SemaphoreType.DMA,
            pltpu.SemaphoreType.DMA,
            pltpu.SemaphoreType.DMA,
            pltpu.SemaphoreType.DMA,
            pltpu.SemaphoreType.DMA,
        ],
    )
    def _sc_agg(h_hbm, epk_hbm, zinit_hbm, out_hbm,
                ebuf, r0, r1, r2, agg_sh, g0, g1, g2, s0, s1, s2, i0, i1):
        rbufs = [r0, r1, r2]
        gsems = [g0, g1, g2]
        ssems = [s0, s1, s2]
        isems = [i0, i1]
        c = lax.axis_index("c")
        s = lax.axis_index("s")
        wid = s * NC + c
        my_epk = epk_hbm.at[wid]
        icp = pltpu.async_copy(my_epk.at[pl.ds(0, NBUF)], ebuf.at[0],
                               isems[0])

        @pl.when(s < NS - 1)
        def _():
            pltpu.sync_copy(zinit_hbm, agg_sh.at[pl.ds(s * WRT, WRT)])

        @pl.when(s == NS - 1)
        def _():
            pltpu.sync_copy(zinit_hbm.at[pl.ds(0, WRT_LAST)],
                            agg_sh.at[pl.ds((NS - 1) * WRT, WRT_LAST)])

        icp.wait()
        plsc.subcore_barrier()

        def body(g, carry):
            slot = lax.rem(g, 2)
            for ss in range(2):
                @pl.when(slot == ss)
                def _():
                    @pl.when(g > 0)
                    def _():
                        for b in range(NBUF):
                            pltpu.make_async_copy(
                                rbufs[b], agg_sh.at[ebuf.at[1 - ss, b, 1]],
                                ssems[b]).wait()

                    @pl.when(g + 1 < NG)
                    def _():
                        pltpu.async_copy(
                            my_epk.at[pl.ds((g + 1) * NBUF, NBUF)],
                            ebuf.at[1 - ss], isems[1 - ss])

                    for b in range(NBUF):
                        pltpu.async_copy(h_hbm.at[ebuf.at[ss, b, 0]],
                                         rbufs[b], gsems[b])

                    for b in range(NBUF):
                        pltpu.make_async_copy(h_hbm.at[ebuf.at[ss, b, 0]],
                                              rbufs[b], gsems[b]).wait()
                        pltpu.async_copy(rbufs[b],
                                         agg_sh.at[ebuf.at[ss, b, 1]],
                                         ssems[b], add=True)

                    @pl.when(g + 1 < NG)
                    def _():
                        pltpu.make_async_copy(
                            my_epk.at[pl.ds((g + 1) * NBUF, NBUF)],
                            ebuf.at[1 - ss], isems[1 - ss]).wait()
            return carry

        lax.fori_loop(0, NG, body, 0)
        for b in range(NBUF):
            pltpu.make_async_copy(rbufs[b], agg_sh.at[ebuf.at[0, b, 1]],
                                  ssems[b]).wait()
        plsc.subcore_barrier()

        @pl.when(s < NS - 1)
        def _():
            pltpu.sync_copy(agg_sh.at[pl.ds(s * WRT, WRT)],
                            out_hbm.at[c].at[pl.ds(s * WRT, WRT)])

        @pl.when(s == NS - 1)
        def _():
            pltpu.sync_copy(agg_sh.at[pl.ds((NS - 1) * WRT, WRT_LAST)],
                            out_hbm.at[c].at[pl.ds((NS - 1) * WRT, WRT_LAST)])

    return _sc_agg


def _tc_layer1_body(part, h, w1, b1, w2, b2, hout):
    agg = part[0] + part[1] + h[...]
    h1 = jnp.maximum(jnp.dot(agg, w1[...], preferred_element_type=jnp.float32)
                     + b1[...], 0.0)
    h2 = jnp.maximum(jnp.dot(h1, w2[...], preferred_element_type=jnp.float32)
                     + b2[...], 0.0)
    hout[...] = h2


def _tc_layer2_body(part, h, w1, b1, w2, b2, bat, wp1, bp1, wp2, bp2,
                    ph_out, out2, pacc1, pacc):
    i = pl.program_id(0)
    agg = part[0] + part[1] + h[...]
    h1 = jnp.maximum(jnp.dot(agg, w1[...], preferred_element_type=jnp.float32)
                     + b1[...], 0.0)
    h2 = jnp.maximum(jnp.dot(h1, w2[...], preferred_element_type=jnp.float32)
                     + b2[...], 0.0)
    oh = (bat[0] == lax.broadcasted_iota(jnp.int32, (G, RB), 0)
          ).astype(jnp.float32)
    contrib1 = jnp.dot(oh, h[...], preferred_element_type=jnp.float32)
    contrib = jnp.dot(oh, h2, preferred_element_type=jnp.float32)

    @pl.when(i == 0)
    def _init():
        pacc1[...] = jnp.zeros((G, G), jnp.float32)
        pacc[...] = jnp.zeros((G, G), jnp.float32)

    pacc1[...] += contrib1
    pacc[...] += contrib

    @pl.when(i == NBLK - 1)
    def _finish():
        ph = jnp.concatenate([pacc1[...], pacc[...]], axis=-1)
        p = jnp.maximum(jnp.dot(ph, wp1[...],
                                preferred_element_type=jnp.float32)
                        + bp1[...], 0.0)
        out2[...] = jnp.dot(p, wp2[...],
                            preferred_element_type=jnp.float32) + bp2[...]
        ph_out[...] = ph


def _tc_layer1(part, h, w1, b1, w2, b2):
    return pl.pallas_call(
        _tc_layer1_body,
        grid=(NBLK,),
        in_specs=[
            pl.BlockSpec((2, RB, D), lambda i: (0, i, 0)),
            pl.BlockSpec((RB, D), lambda i: (i, 0)),
            pl.BlockSpec((D, D), lambda i: (0, 0)),
            pl.BlockSpec((1, D), lambda i: (0, 0)),
            pl.BlockSpec((D, D), lambda i: (0, 0)),
            pl.BlockSpec((1, D), lambda i: (0, 0)),
        ],
        out_specs=pl.BlockSpec((RB, D), lambda i: (i, 0)),
        out_shape=jax.ShapeDtypeStruct((N, D), jnp.float32),
    )(part, h, w1, b1, w2, b2)


def _tc_layer2(part, h, w1, b1, w2, b2, bat3, wp1, bp1, wp2, bp2):
    return pl.pallas_call(
        _tc_layer2_body,
        grid=(NBLK,),
        in_specs=[
            pl.BlockSpec((2, RB, D), lambda i: (0, i, 0)),
            pl.BlockSpec((RB, D), lambda i: (i, 0)),
            pl.BlockSpec((D, D), lambda i: (0, 0)),
            pl.BlockSpec((1, D), lambda i: (0, 0)),
            pl.BlockSpec((D, D), lambda i: (0, 0)),
            pl.BlockSpec((1, D), lambda i: (0, 0)),
            pl.BlockSpec((1, 1, RB), lambda i: (i, 0, 0)),
            pl.BlockSpec((2 * D, D), lambda i: (0, 0)),
            pl.BlockSpec((1, D), lambda i: (0, 0)),
            pl.BlockSpec((D, D), lambda i: (0, 0)),
            pl.BlockSpec((1, D), lambda i: (0, 0)),
        ],
        out_specs=[
            pl.BlockSpec((G, 2 * D), lambda i: (0, 0)),
            pl.BlockSpec((G, D), lambda i: (0, 0)),
        ],
        out_shape=[
            jax.ShapeDtypeStruct((G, 2 * D), jnp.float32),
            jax.ShapeDtypeStruct((G, D), jnp.float32),
        ],
        scratch_shapes=[pltpu.VMEM((G, G), jnp.float32),
                        pltpu.VMEM((G, G), jnp.float32)],
    )(part, h, w1, b1, w2, b2, bat3, wp1, bp1, wp2, bp2)


def kernel(x, edge_index, batch, W1_0, b1_0, W2_0, b2_0, W1_1, b1_1,
           W2_1, b2_1, Wp1, bp1, Wp2, bp2):
    src = edge_index[0].astype(jnp.int32)
    dst = edge_index[1].astype(jnp.int32)

    ar = jnp.arange(PAD_E, dtype=jnp.int32)[None, :]
    w = jnp.arange(NW, dtype=jnp.int32)[:, None]
    pad_src = (w * 313 + ar) % N
    pad_dst = N + (w * 8 + ar) % (AGG_ROWS - N)
    srcp = jnp.concatenate([src.reshape(NW, EPT), pad_src],
                           axis=1).reshape(NW, NCHUNK, CH)
    dstp = jnp.concatenate([dst.reshape(NW, EPT), pad_dst],
                           axis=1).reshape(NW, NCHUNK, CH)
    epk = jnp.stack([srcp, dstp], axis=2)
    zinit = jnp.zeros((WRT, D), jnp.float32)
    bat3 = batch.astype(jnp.int32).reshape(NBLK, 1, RB)

    b1_0r, b2_0r = b1_0.reshape(1, D), b2_0.reshape(1, D)
    b1_1r, b2_1r = b1_1.reshape(1, D), b2_1.reshape(1, D)
    bp1r, bp2r = bp1.reshape(1, D), bp2.reshape(1, D)

    sc_agg = _make_sc_agg()
    part1 = sc_agg(x, epk, zinit)
    h1 = _tc_layer1(part1, x, W1_0, b1_0r, W2_0, b2_0r)
    part2 = sc_agg(h1, epk, zinit)
    pooled_h, pooled_h_p = _tc_layer2(part2, h1, W1_1, b1_1r, W2_1, b2_1r,
                                      bat3, Wp1, bp1r, Wp2, bp2r)
    return (pooled_h, pooled_h_p, x)


# speedup vs baseline: 1.0263x; 1.0226x over previous
import functools

import jax
import jax.numpy as jnp
from jax import lax
from jax.experimental import pallas as pl
from jax.experimental.pallas import tpu as pltpu
from jax.experimental.pallas import tpu_sc as plsc

N = 10000
D = 128
E = 320000
G = 128
NC, NS = 2, 16
NW = NC * NS
CH = 128
EPT = E // NW
NCHUNK = 81
EPT_PAD = NCHUNK * CH
PAD_E = EPT_PAD - EPT
AGG_ROWS = 10040
WRT = 632
WRT_LAST = AGG_ROWS - 15 * WRT

RB = 2000
NBLK = N // RB


NBUF = 3


@functools.lru_cache(maxsize=None)
def _make_sc_agg():
    mesh = plsc.VectorSubcoreMesh(core_axis_name="c", subcore_axis_name="s")

    NG = NCHUNK // NBUF

    @functools.partial(
        pl.kernel,
        mesh=mesh,
        out_type=jax.ShapeDtypeStruct((NC, AGG_ROWS, D), jnp.float32),
        scratch_types=[
            pltpu.VMEM((2, NBUF, 2, CH), jnp.int32),
            pltpu.VMEM((CH, D), jnp.float32),
            pltpu.VMEM((CH, D), jnp.float32),
            pltpu.VMEM((CH, D), jnp.float32),
            pltpu.VMEM_SHARED((AGG_ROWS, D), jnp.float32),
            pltpu.SemaphoreType.DMA,
            pltpu.SemaphoreType.DMA,
            pltpu.SemaphoreType.DMA,
            pltpu.SemaphoreType.DMA,
            pltpu.SemaphoreType.DMA,
            pltpu.SemaphoreType.DMA,
            pltpu.SemaphoreType.DMA,
            pltpu.SemaphoreType.DMA,
        ],
    )
    def _sc_agg(h_hbm, epk_hbm, out_hbm,
                ebuf, r0, r1, r2, agg_sh, g0, g1, g2, s0, s1, s2, i0, i1):
        rbufs = [r0, r1, r2]
        gsems = [g0, g1, g2]
        ssems = [s0, s1, s2]
        isems = [i0, i1]
        c = lax.axis_index("c")
        s = lax.axis_index("s")
        wid = s * NC + c
        my_epk = epk_hbm.at[wid]
        icp = pltpu.async_copy(my_epk.at[pl.ds(0, NBUF)], ebuf.at[0],
                               isems[0])

        def zrow(r, carry):
            for cc in range(D // 16):
                r0[r, pl.ds(cc * 16, 16)] = jnp.zeros((16,), jnp.float32)
            return carry

        lax.fori_loop(0, CH, zrow, 0)
        zbase = s * WRT
        for k in range(4):
            pltpu.sync_copy(r0, agg_sh.at[pl.ds(zbase + k * CH, CH)])

        @pl.when(s < NS - 1)
        def _():
            pltpu.sync_copy(r0.at[pl.ds(0, WRT - 4 * CH)],
                            agg_sh.at[pl.ds(zbase + 4 * CH, WRT - 4 * CH)])

        @pl.when(s == NS - 1)
        def _():
            pltpu.sync_copy(r0.at[pl.ds(0, WRT_LAST - 4 * CH)],
                            agg_sh.at[pl.ds(zbase + 4 * CH,
                                            WRT_LAST - 4 * CH)])

        icp.wait()
        plsc.subcore_barrier()

        def body(g, carry):
            slot = lax.rem(g, 2)
            for ss in range(2):
                @pl.when(slot == ss)
                def _():
                    @pl.when(g > 0)
                    def _():
                        for b in range(NBUF):
                            pltpu.make_async_copy(
                                rbufs[b], agg_sh.at[ebuf.at[1 - ss, b, 1]],
                                ssems[b]).wait()

                    @pl.when(g + 1 < NG)
                    def _():
                        pltpu.async_copy(
                            my_epk.at[pl.ds((g + 1) * NBUF, NBUF)],
                            ebuf.at[1 - ss], isems[1 - ss])

                    for b in range(NBUF):
                        pltpu.async_copy(h_hbm.at[ebuf.at[ss, b, 0]],
                                         rbufs[b], gsems[b])

                    for b in range(NBUF):
                        pltpu.make_async_copy(h_hbm.at[ebuf.at[ss, b, 0]],
                                              rbufs[b], gsems[b]).wait()
                        pltpu.async_copy(rbufs[b],
                                         agg_sh.at[ebuf.at[ss, b, 1]],
                                         ssems[b], add=True)

                    @pl.when(g + 1 < NG)
                    def _():
                        pltpu.make_async_copy(
                            my_epk.at[pl.ds((g + 1) * NBUF, NBUF)],
                            ebuf.at[1 - ss], isems[1 - ss]).wait()
            return carry

        lax.fori_loop(0, NG, body, 0)
        for b in range(NBUF):
            pltpu.make_async_copy(rbufs[b], agg_sh.at[ebuf.at[0, b, 1]],
                                  ssems[b]).wait()
        plsc.subcore_barrier()

        @pl.when(s < NS - 1)
        def _():
            pltpu.sync_copy(agg_sh.at[pl.ds(s * WRT, WRT)],
                            out_hbm.at[c].at[pl.ds(s * WRT, WRT)])

        @pl.when(s == NS - 1)
        def _():
            pltpu.sync_copy(agg_sh.at[pl.ds((NS - 1) * WRT, WRT_LAST)],
                            out_hbm.at[c].at[pl.ds((NS - 1) * WRT, WRT_LAST)])

    return _sc_agg


def _tc_layer1_body(part, h, w1, b1, w2, b2, hout):
    agg = part[0] + part[1] + h[...]
    h1 = jnp.maximum(jnp.dot(agg, w1[...], preferred_element_type=jnp.float32)
                     + b1[...], 0.0)
    h2 = jnp.maximum(jnp.dot(h1, w2[...], preferred_element_type=jnp.float32)
                     + b2[...], 0.0)
    hout[...] = h2


def _tc_layer2_body(part, h, w1, b1, w2, b2, bat, wp1, bp1, wp2, bp2,
                    ph_out, out2, pacc1, pacc):
    i = pl.program_id(0)
    agg = part[0] + part[1] + h[...]
    h1 = jnp.maximum(jnp.dot(agg, w1[...], preferred_element_type=jnp.float32)
                     + b1[...], 0.0)
    h2 = jnp.maximum(jnp.dot(h1, w2[...], preferred_element_type=jnp.float32)
                     + b2[...], 0.0)
    oh = (bat[0] == lax.broadcasted_iota(jnp.int32, (G, RB), 0)
          ).astype(jnp.float32)
    contrib1 = jnp.dot(oh, h[...], preferred_element_type=jnp.float32)
    contrib = jnp.dot(oh, h2, preferred_element_type=jnp.float32)

    @pl.when(i == 0)
    def _init():
        pacc1[...] = jnp.zeros((G, G), jnp.float32)
        pacc[...] = jnp.zeros((G, G), jnp.float32)

    pacc1[...] += contrib1
    pacc[...] += contrib

    @pl.when(i == NBLK - 1)
    def _finish():
        ph = jnp.concatenate([pacc1[...], pacc[...]], axis=-1)
        p = jnp.maximum(jnp.dot(ph, wp1[...],
                                preferred_element_type=jnp.float32)
                        + bp1[...], 0.0)
        out2[...] = jnp.dot(p, wp2[...],
                            preferred_element_type=jnp.float32) + bp2[...]
        ph_out[...] = ph


def _tc_layer1(part, h, w1, b1, w2, b2):
    return pl.pallas_call(
        _tc_layer1_body,
        grid=(NBLK,),
        in_specs=[
            pl.BlockSpec((2, RB, D), lambda i: (0, i, 0)),
            pl.BlockSpec((RB, D), lambda i: (i, 0)),
            pl.BlockSpec((D, D), lambda i: (0, 0)),
            pl.BlockSpec((1, D), lambda i: (0, 0)),
            pl.BlockSpec((D, D), lambda i: (0, 0)),
            pl.BlockSpec((1, D), lambda i: (0, 0)),
        ],
        out_specs=pl.BlockSpec((RB, D), lambda i: (i, 0)),
        out_shape=jax.ShapeDtypeStruct((N, D), jnp.float32),
    )(part, h, w1, b1, w2, b2)


def _tc_layer2(part, h, w1, b1, w2, b2, bat3, wp1, bp1, wp2, bp2):
    return pl.pallas_call(
        _tc_layer2_body,
        grid=(NBLK,),
        in_specs=[
            pl.BlockSpec((2, RB, D), lambda i: (0, i, 0)),
            pl.BlockSpec((RB, D), lambda i: (i, 0)),
            pl.BlockSpec((D, D), lambda i: (0, 0)),
            pl.BlockSpec((1, D), lambda i: (0, 0)),
            pl.BlockSpec((D, D), lambda i: (0, 0)),
            pl.BlockSpec((1, D), lambda i: (0, 0)),
            pl.BlockSpec((1, 1, RB), lambda i: (i, 0, 0)),
            pl.BlockSpec((2 * D, D), lambda i: (0, 0)),
            pl.BlockSpec((1, D), lambda i: (0, 0)),
            pl.BlockSpec((D, D), lambda i: (0, 0)),
            pl.BlockSpec((1, D), lambda i: (0, 0)),
        ],
        out_specs=[
            pl.BlockSpec((G, 2 * D), lambda i: (0, 0)),
            pl.BlockSpec((G, D), lambda i: (0, 0)),
        ],
        out_shape=[
            jax.ShapeDtypeStruct((G, 2 * D), jnp.float32),
            jax.ShapeDtypeStruct((G, D), jnp.float32),
        ],
        scratch_shapes=[pltpu.VMEM((G, G), jnp.float32),
                        pltpu.VMEM((G, G), jnp.float32)],
    )(part, h, w1, b1, w2, b2, bat3, wp1, bp1, wp2, bp2)


def kernel(x, edge_index, batch, W1_0, b1_0, W2_0, b2_0, W1_1, b1_1,
           W2_1, b2_1, Wp1, bp1, Wp2, bp2):
    src = edge_index[0].astype(jnp.int32)
    dst = edge_index[1].astype(jnp.int32)

    ar = jnp.arange(PAD_E, dtype=jnp.int32)[None, :]
    w = jnp.arange(NW, dtype=jnp.int32)[:, None]
    pad_src = (w * 313 + ar) % N
    pad_dst = N + (w * 8 + ar) % (AGG_ROWS - N)
    srcp = jnp.concatenate([src.reshape(NW, EPT), pad_src],
                           axis=1).reshape(NW, NCHUNK, CH)
    dstp = jnp.concatenate([dst.reshape(NW, EPT), pad_dst],
                           axis=1).reshape(NW, NCHUNK, CH)
    epk = jnp.stack([srcp, dstp], axis=2)
    bat3 = batch.astype(jnp.int32).reshape(NBLK, 1, RB)

    b1_0r, b2_0r = b1_0.reshape(1, D), b2_0.reshape(1, D)
    b1_1r, b2_1r = b1_1.reshape(1, D), b2_1.reshape(1, D)
    bp1r, bp2r = bp1.reshape(1, D), bp2.reshape(1, D)

    sc_agg = _make_sc_agg()
    part1 = sc_agg(x, epk)
    h1 = _tc_layer1(part1, x, W1_0, b1_0r, W2_0, b2_0r)
    part2 = sc_agg(h1, epk)
    pooled_h, pooled_h_p = _tc_layer2(part2, h1, W1_1, b1_1r, W2_1, b2_1r,
                                      bat3, Wp1, bp1r, Wp2, bp2r)
    return (pooled_h, pooled_h_p, x)
